# baseline (speedup 1.0000x reference)
import functools

import jax
import jax.numpy as jnp
from jax import lax
from jax.experimental import pallas as pl
from jax.experimental.pallas import tpu as pltpu
from jax.experimental.pallas import tpu_sc as plsc

_NCORE = 2
_NSUB = 16
_NT = _NCORE * _NSUB
_K = 80
_HALF = 5120
_NPAD = 10240
_CAP = 10240


def _mm_body(alpha_ref, comp_ref, x_ref, w_ref, upd_ref, o1_ref, o2_ref):
    r = pl.program_id(1)
    x = x_ref[...]

    @pl.when(r < 16)
    def _():
        w = (comp_ref[r, 0] * w_ref[0] + comp_ref[r, 1] * w_ref[1]
             + comp_ref[r, 2] * w_ref[2] + comp_ref[r, 3] * w_ref[3])
        o1_ref[0] = (alpha_ref[r, 0] * jnp.dot(
            x, w, preferred_element_type=jnp.float32)).astype(jnp.bfloat16)

    @pl.when(r == 16)
    def _():
        o2_ref[...] = jnp.dot(
            x, upd_ref[...], preferred_element_type=jnp.float32)


def _build_tables(alphas, comp, vfts, weight, update):
    n, d = vfts.shape
    nrel = comp.shape[0]
    nblk = 10
    b = n // nblk
    return pl.pallas_call(
        _mm_body,
        grid=(nblk, nrel + 1),
        in_specs=[
            pl.BlockSpec((nrel, 1), lambda i, r: (0, 0),
                         memory_space=pltpu.SMEM),
            pl.BlockSpec((nrel, 4), lambda i, r: (0, 0),
                         memory_space=pltpu.SMEM),
            pl.BlockSpec((b, d), lambda i, r: (i, 0)),
            pl.BlockSpec((4, d, d), lambda i, r: (0, 0, 0)),
            pl.BlockSpec((d, d), lambda i, r: (0, 0)),
        ],
        out_specs=[
            pl.BlockSpec((1, b, d), lambda i, r: (jnp.minimum(r, 15), i, 0)),
            pl.BlockSpec((b, d), lambda i, r: (i, 0)),
        ],
        out_shape=[
            jax.ShapeDtypeStruct((nrel, n, d), jnp.bfloat16),
            jax.ShapeDtypeStruct((n, d), jnp.float32),
        ],
        compiler_params=pltpu.CompilerParams(
            dimension_semantics=("arbitrary", "arbitrary")),
    )(alphas, comp, vfts, weight, update)


def _make_router(n, e):
    ept = e // _NT
    seg = 2000
    nseg = ept // seg
    mesh = plsc.VectorSubcoreMesh(
        core_axis_name="c", subcore_axis_name="s",
        num_cores=_NCORE, num_subcores=_NSUB)

    @functools.partial(
        pl.kernel,
        out_type=(
            jax.ShapeDtypeStruct((_NT, _NCORE, 1, _CAP), jnp.int32),
            jax.ShapeDtypeStruct((_NT, _NCORE, 1, _CAP), jnp.int32),
            jax.ShapeDtypeStruct((_NT, 1, 16), jnp.int32),
            jax.ShapeDtypeStruct((_NT, 1, _NPAD), jnp.float32),
        ),
        mesh=mesh,
        scratch_types=[
            pltpu.VMEM((seg,), jnp.int32),
            pltpu.VMEM((seg,), jnp.int32),
            pltpu.VMEM((seg,), jnp.int32),
            pltpu.VMEM((_CAP,), jnp.int32),
            pltpu.VMEM((_CAP,), jnp.int32),
            pltpu.VMEM((_CAP,), jnp.int32),
            pltpu.VMEM((_CAP,), jnp.int32),
            pltpu.VMEM((_NPAD,), jnp.float32),
            pltpu.VMEM((16,), jnp.int32),
        ],
        compiler_params=pltpu.CompilerParams(needs_layout_passes=False),
    )
    def router(rels_hbm, src_hbm, dst_hbm, zcnt_hbm,
               bg_out, bd_out, len_out, cnt_out,
               rels_v, src_v, dst_v, bg0, bd0, bg1, bd1, cnt_v, lens_v):
        c = lax.axis_index("c")
        s = lax.axis_index("s")
        wid = c * _NSUB + s
        base_e = wid * ept

        pltpu.sync_copy(zcnt_hbm, cnt_v)

        ones16 = jnp.ones((16,), jnp.float32)
        zeros16 = jnp.zeros((16,), jnp.int32)
        nvec = jnp.full((16,), n, jnp.int32)
        hvec = jnp.full((16,), _HALF, jnp.int32)
        trash16 = jnp.full((16,), _HALF, jnp.int32)
        iota16 = jnp.arange(16, dtype=jnp.int32)

        def seg_body(g, offs):
            off = base_e + g * seg
            pltpu.sync_copy(rels_hbm.at[pl.ds(off, seg)], rels_v)
            pltpu.sync_copy(src_hbm.at[pl.ds(off, seg)], src_v)
            pltpu.sync_copy(dst_hbm.at[pl.ds(off, seg)], dst_v)

            def vec_body(i, offs2):
                o0, o1 = offs2
                sl = pl.ds(i * 16, 16)
                dv = dst_v[sl]
                gx = rels_v[sl] * nvec + src_v[sl]
                plsc.addupdate_scatter(cnt_v, [dv], ones16)
                m0 = dv < hvec
                m1 = jnp.logical_not(m0)
                plsc.store_compressed(bg0.at[pl.ds(o0, 16)], gx, mask=m0)
                plsc.store_compressed(bd0.at[pl.ds(o0, 16)], dv, mask=m0)
                plsc.store_compressed(bg1.at[pl.ds(o1, 16)], gx, mask=m1)
                plsc.store_compressed(bd1.at[pl.ds(o1, 16)], dv - hvec,
                                      mask=m1)
                pc = jnp.max(plsc.all_reduce_population_count(m0))
                return (o0 + pc, o1 + (16 - pc))

            return lax.fori_loop(0, seg // 16, vec_body, offs)

        z = jnp.int32(0)
        off0, off1 = lax.fori_loop(0, nseg, seg_body, (z, z))

        for k in range(20):
            pad = k * 16

            @pl.when(off0 + pad < _CAP)
            def _():
                bg0[pl.ds(off0 + pad, 16)] = zeros16
                bd0[pl.ds(off0 + pad, 16)] = trash16

            @pl.when(off1 + pad < _CAP)
            def _():
                bg1[pl.ds(off1 + pad, 16)] = zeros16
                bd1[pl.ds(off1 + pad, 16)] = trash16

        lens_v[...] = jnp.where(
            iota16 == 0, off0, jnp.where(iota16 == 1, off1, 0))

        pltpu.sync_copy(bg0, bg_out.at[wid, 0, 0])
        pltpu.sync_copy(bd0, bd_out.at[wid, 0, 0])
        pltpu.sync_copy(bg1, bg_out.at[wid, 1, 0])
        pltpu.sync_copy(bd1, bd_out.at[wid, 1, 0])
        pltpu.sync_copy(lens_v, len_out.at[wid, 0])
        pltpu.sync_copy(cnt_v, cnt_out.at[wid, 0])

    return router


def _make_accum(n, d):
    rpt = _HALF // _NSUB
    nko = rpt // _K
    nsrc = _NT // _NSUB
    mesh = plsc.VectorSubcoreMesh(
        core_axis_name="c", subcore_axis_name="s",
        num_cores=_NCORE, num_subcores=_NSUB)

    @functools.partial(
        pl.kernel,
        out_type=jax.ShapeDtypeStruct((_NCORE, _HALF, d), jnp.bfloat16),
        mesh=mesh,
        scratch_types=[
            pltpu.VMEM((_CAP,), jnp.int32),
            pltpu.VMEM((_CAP,), jnp.int32),
            pltpu.VMEM((_CAP // _K, _K), jnp.int32),
            pltpu.VMEM((16,), jnp.int32),
            pltpu.VMEM((4, _K, d), jnp.bfloat16),
            pltpu.VMEM_SHARED((_HALF + 8, d), jnp.bfloat16),
            pltpu.SemaphoreType.DMA,
            pltpu.SemaphoreType.DMA,
            pltpu.SemaphoreType.DMA,
            pltpu.SemaphoreType.DMA,
        ],
        compiler_params=pltpu.CompilerParams(
            needs_layout_passes=False, use_tc_tiling_on_sc=False),
    )
    def accum(tbl_hbm, bg_hbm, bd_hbm, len_hbm, zrow_hbm,
              acc_out, lg_v, ld_v, ld2_v, lens_v, rows_v, acc_sh,
              *sems):
        gsem = sems
        c = lax.axis_index("c")
        s = lax.axis_index("s")

        pltpu.sync_copy(zrow_hbm, rows_v.at[1])
        for k in range(nko):
            pltpu.sync_copy(rows_v.at[1],
                            acc_sh.at[pl.ds(s * rpt + k * _K, _K)])

        @pl.when(s == 0)
        def _():
            pltpu.sync_copy(rows_v.at[1, pl.ds(0, 8)],
                            acc_sh.at[pl.ds(_HALF, 8)])

        plsc.subcore_barrier()

        iota16 = jnp.arange(16, dtype=jnp.int32)
        cvec = jnp.zeros((16,), jnp.int32) + c

        for tt in range(nsrc):
            t = nsrc * s + tt
            pltpu.sync_copy(bg_hbm.at[t, c, 0], lg_v)
            pltpu.sync_copy(bd_hbm.at[t, c, 0], ld_v)
            pltpu.sync_copy(len_hbm.at[t, 0], lens_v)
            ln = jnp.max(jnp.where(iota16 == cvec, lens_v[...], 0))
            nquad = (ln + 4 * _K - 1) // (4 * _K)
            ntot = nquad * 4

            def mv_body(i, cr):
                row = i // (_K // 16)
                col = (i % (_K // 16)) * 16
                ld2_v[row, pl.ds(col, 16)] = ld_v[pl.ds(i * 16, 16)]
                return cr

            lax.fori_loop(0, nquad * 4 * (_K // 16), mv_body, 0)

            @pl.when(nquad > 0)
            def _():
                for b in range(4):
                    pltpu.async_copy(
                        tbl_hbm.at[lg_v.at[pl.ds(b * _K, _K)]],
                        rows_v.at[b], gsem[b])

            def quad_body(q, cr):
                j = 4 * q
                for b in range(4):
                    pltpu.make_async_copy(
                        tbl_hbm.at[pl.ds(0, _K)], rows_v.at[b],
                        gsem[b]).wait()
                    pltpu.sync_copy(rows_v.at[b],
                                    acc_sh.at[ld2_v.at[j + b]], add=True)

                    @pl.when(j + b + 4 < ntot)
                    def _(b=b, j=j):
                        pltpu.async_copy(
                            tbl_hbm.at[lg_v.at[pl.ds((j + b + 4) * _K, _K)]],
                            rows_v.at[b], gsem[b])
                return cr

            lax.fori_loop(0, nquad, quad_body, 0)

        plsc.subcore_barrier()

        for k in range(nko):
            pltpu.sync_copy(acc_sh.at[pl.ds(s * rpt + k * _K, _K)],
                            rows_v.at[1])
            pltpu.sync_copy(rows_v.at[1],
                            acc_out.at[c, pl.ds(s * rpt + k * _K, _K)])

    return accum


def _combine_body(b, acc_ref, cnt_ref, dst_ref, o_ref):
    i = pl.program_id(0)
    deg = jnp.sum(cnt_ref[:, pl.ds(i * b, b)], axis=0)
    acc32 = acc_ref[...].astype(jnp.float32)
    o_ref[...] = jnp.maximum(acc32 + deg[:, None] * dst_ref[...], 0.0)


def _combine(acc, cnt, dsts):
    n, d = dsts.shape
    nt = cnt.shape[0]
    b = 512
    nblk = (n + b - 1) // b
    return pl.pallas_call(
        functools.partial(_combine_body, b),
        grid=(nblk,),
        in_specs=[
            pl.BlockSpec((b, d), lambda i: (i, 0)),
            pl.BlockSpec((nt, _NPAD), lambda i: (0, 0)),
            pl.BlockSpec((b, d), lambda i: (i, 0)),
        ],
        out_specs=pl.BlockSpec((b, d), lambda i: (i, 0)),
        out_shape=jax.ShapeDtypeStruct((n, d), jnp.float32),
    )(acc, cnt, dsts)


def kernel(vfts, adjs, rels, embed_rels, embed_rels_target, weight, comp,
           attn_w1, attn_b1, attn_w2, attn_b2, update):
    n, d = vfts.shape
    e = rels.shape[0]
    nrel = comp.shape[0]
    assert e % (_NT * 2000) == 0 and _NCORE * _HALF >= n

    s16 = adjs[0, :nrel]
    t16 = adjs[1, :nrel]
    erps16 = jnp.concatenate(
        [vfts[s16], vfts[t16], embed_rels[:nrel],
         embed_rels_target[:nrel]], axis=1)
    h16 = jax.nn.relu(erps16 @ attn_w1.T + attn_b1)
    alphas = jax.nn.sigmoid(h16 @ attn_w2.T + attn_b2)

    tbl3, dsts = _build_tables(alphas, comp, vfts, weight, update)
    tbl = tbl3.reshape(nrel * n, d)

    zrow = jnp.zeros((_K, d), jnp.bfloat16)
    zcnt = jnp.zeros((_NPAD,), jnp.float32)

    bg, bd, lens, cnt = _make_router(n, e)(rels, adjs[0], adjs[1], zcnt)
    acc = _make_accum(n, d)(tbl, bg, bd, lens, zrow)
    return _combine(acc.reshape(_NCORE * _HALF, d),
                    cnt.reshape(_NT, _NPAD), dsts)


# speedup vs baseline: 1.1739x; 1.1739x over previous
import functools

import jax
import jax.numpy as jnp
from jax import lax
from jax.experimental import pallas as pl
from jax.experimental.pallas import tpu as pltpu
from jax.experimental.pallas import tpu_sc as plsc

_NCORE = 2
_NSUB = 16
_K = 80
_HALF = 5120
_NPAD = 10240


def _mm_body(alpha_ref, comp_ref, x_ref, w_ref, upd_ref, o1_ref, o2_ref):
    r = pl.program_id(1)
    x = x_ref[...]

    @pl.when(r < 16)
    def _():
        w = (comp_ref[r, 0] * w_ref[0] + comp_ref[r, 1] * w_ref[1]
             + comp_ref[r, 2] * w_ref[2] + comp_ref[r, 3] * w_ref[3])
        o1_ref[0] = (alpha_ref[r, 0] * jnp.dot(
            x, w, preferred_element_type=jnp.float32)).astype(jnp.bfloat16)

    @pl.when(r == 16)
    def _():
        o2_ref[...] = jnp.dot(
            x, upd_ref[...], preferred_element_type=jnp.float32)


def _build_tables(alphas, comp, vfts, weight, update):
    n, d = vfts.shape
    nrel = comp.shape[0]
    nblk = 10
    b = n // nblk
    return pl.pallas_call(
        _mm_body,
        grid=(nblk, nrel + 1),
        in_specs=[
            pl.BlockSpec((nrel, 1), lambda i, r: (0, 0),
                         memory_space=pltpu.SMEM),
            pl.BlockSpec((nrel, 4), lambda i, r: (0, 0),
                         memory_space=pltpu.SMEM),
            pl.BlockSpec((b, d), lambda i, r: (i, 0)),
            pl.BlockSpec((4, d, d), lambda i, r: (0, 0, 0)),
            pl.BlockSpec((d, d), lambda i, r: (0, 0)),
        ],
        out_specs=[
            pl.BlockSpec((1, b, d), lambda i, r: (jnp.minimum(r, 15), i, 0)),
            pl.BlockSpec((b, d), lambda i, r: (i, 0)),
        ],
        out_shape=[
            jax.ShapeDtypeStruct((nrel, n, d), jnp.bfloat16),
            jax.ShapeDtypeStruct((n, d), jnp.float32),
        ],
        compiler_params=pltpu.CompilerParams(
            dimension_semantics=("arbitrary", "arbitrary")),
    )(alphas, comp, vfts, weight, update)


_SEG = 50


def _make_sc(n, d, e):
    ept = e // _NSUB
    seg = _SEG * _K
    nseg = ept // seg
    rpt = _HALF // _NSUB
    nko = rpt // _K
    mesh = plsc.VectorSubcoreMesh(
        core_axis_name="c", subcore_axis_name="s",
        num_cores=_NCORE, num_subcores=_NSUB)

    @functools.partial(
        pl.kernel,
        out_type=(
            jax.ShapeDtypeStruct((_NCORE, _HALF, d), jnp.bfloat16),
            jax.ShapeDtypeStruct((_NSUB, 1, _NPAD), jnp.float32),
        ),
        mesh=mesh,
        scratch_types=[
            pltpu.VMEM((seg,), jnp.int32),
            pltpu.VMEM((seg,), jnp.int32),
            pltpu.VMEM((seg,), jnp.int32),
            pltpu.VMEM((seg,), jnp.int32),
            pltpu.VMEM((_SEG, _K), jnp.int32),
            pltpu.VMEM((_NPAD,), jnp.float32),
            pltpu.VMEM((2, _K, d), jnp.bfloat16),
            pltpu.VMEM_SHARED((_HALF + 8, d), jnp.bfloat16),
            pltpu.SemaphoreType.DMA,
            pltpu.SemaphoreType.DMA,
        ],
        compiler_params=pltpu.CompilerParams(
            needs_layout_passes=False, use_tc_tiling_on_sc=False),
    )
    def sc_kernel(tbl_hbm, rels_hbm, src_hbm, dst_hbm, zrow_hbm, zcnt_hbm,
                  acc_out, cnt_out,
                  rels_v, src_v, gidx_v, dst1_v, dst2_v, cnt_v, rows_v,
                  acc_sh, sem0, sem1):
        c = lax.axis_index("c")
        s = lax.axis_index("s")
        base_e = s * ept

        pltpu.sync_copy(zcnt_hbm, cnt_v)

        pltpu.sync_copy(zrow_hbm, rows_v.at[1])
        for k in range(nko):
            pltpu.sync_copy(rows_v.at[1],
                            acc_sh.at[pl.ds(s * rpt + k * _K, _K)])

        @pl.when(s == 0)
        def _():
            pltpu.sync_copy(rows_v.at[1, pl.ds(0, 8)],
                            acc_sh.at[pl.ds(_HALF, 8)])

        plsc.subcore_barrier()

        ones16 = jnp.ones((16,), jnp.float32)
        nvec = jnp.full((16,), n, jnp.int32)
        lovec = jnp.zeros((16,), jnp.int32) + c * _HALF
        trash = jnp.full((16,), _HALF, jnp.int32)

        def seg_body(g, carry):
            off = base_e + g * seg
            pltpu.sync_copy(rels_hbm.at[pl.ds(off, seg)], rels_v)
            pltpu.sync_copy(src_hbm.at[pl.ds(off, seg)], src_v)
            pltpu.sync_copy(dst_hbm.at[pl.ds(off, seg)], dst1_v)

            def idx_body(i, cr):
                sl = pl.ds(i * 16, 16)
                gidx_v[sl] = rels_v[sl] * nvec + src_v[sl]
                dv = dst1_v[sl]

                @pl.when(c == 0)
                def _():
                    plsc.addupdate_scatter(cnt_v, [dv], ones16)

                dl = dv - lovec
                oob = (dl < 0) | (dl >= _HALF)
                dst2_v[i // (_K // 16),
                       pl.ds((i % (_K // 16)) * 16, 16)] = jnp.where(
                           oob, trash, dl)
                return cr

            lax.fori_loop(0, seg // 16, idx_body, 0)

            def pair_body(jj, cr):
                j0 = 2 * jj
                j1 = j0 + 1
                g0 = pltpu.async_copy(
                    tbl_hbm.at[gidx_v.at[pl.ds(j0 * _K, _K)]],
                    rows_v.at[0], sem0)
                g1 = pltpu.async_copy(
                    tbl_hbm.at[gidx_v.at[pl.ds(j1 * _K, _K)]],
                    rows_v.at[1], sem1)
                g0.wait()
                pltpu.sync_copy(rows_v.at[0], acc_sh.at[dst2_v.at[j0]],
                                add=True)
                g1.wait()
                pltpu.sync_copy(rows_v.at[1], acc_sh.at[dst2_v.at[j1]],
                                add=True)
                return cr

            lax.fori_loop(0, _SEG // 2, pair_body, 0)
            return carry

        lax.fori_loop(0, nseg, seg_body, 0)

        plsc.subcore_barrier()

        for k in range(nko):
            pltpu.sync_copy(acc_sh.at[pl.ds(s * rpt + k * _K, _K)],
                            rows_v.at[1])
            pltpu.sync_copy(rows_v.at[1],
                            acc_out.at[c, pl.ds(s * rpt + k * _K, _K)])

        @pl.when(c == 0)
        def _():
            pltpu.sync_copy(cnt_v, cnt_out.at[s, 0])

    return sc_kernel


def _combine_body(b, acc_ref, cnt_ref, dst_ref, o_ref):
    i = pl.program_id(0)
    deg = jnp.sum(cnt_ref[:, pl.ds(i * b, b)], axis=0)
    acc32 = acc_ref[...].astype(jnp.float32)
    o_ref[...] = jnp.maximum(acc32 + deg[:, None] * dst_ref[...], 0.0)


def _combine(acc, cnt, dsts):
    n, d = dsts.shape
    b = 512
    nblk = (n + b - 1) // b
    return pl.pallas_call(
        functools.partial(_combine_body, b),
        grid=(nblk,),
        in_specs=[
            pl.BlockSpec((b, d), lambda i: (i, 0)),
            pl.BlockSpec((_NSUB, _NPAD), lambda i: (0, 0)),
            pl.BlockSpec((b, d), lambda i: (i, 0)),
        ],
        out_specs=pl.BlockSpec((b, d), lambda i: (i, 0)),
        out_shape=jax.ShapeDtypeStruct((n, d), jnp.float32),
    )(acc, cnt, dsts)


def kernel(vfts, adjs, rels, embed_rels, embed_rels_target, weight, comp,
           attn_w1, attn_b1, attn_w2, attn_b2, update):
    n, d = vfts.shape
    e = rels.shape[0]
    nrel = comp.shape[0]
    assert e % (_NSUB * _SEG * _K) == 0 and _NCORE * _HALF >= n

    s16 = adjs[0, :nrel]
    t16 = adjs[1, :nrel]
    erps16 = jnp.concatenate(
        [vfts[s16], vfts[t16], embed_rels[:nrel],
         embed_rels_target[:nrel]], axis=1)
    h16 = jax.nn.relu(erps16 @ attn_w1.T + attn_b1)
    alphas = jax.nn.sigmoid(h16 @ attn_w2.T + attn_b2)

    tbl3, dsts = _build_tables(alphas, comp, vfts, weight, update)
    tbl = tbl3.reshape(nrel * n, d)

    zrow = jnp.zeros((_K, d), jnp.bfloat16)
    zcnt = jnp.zeros((_NPAD,), jnp.float32)

    acc, cnt = _make_sc(n, d, e)(tbl, rels, adjs[0], adjs[1], zrow, zcnt)
    return _combine(acc.reshape(_NCORE * _HALF, d),
                    cnt.reshape(_NSUB, _NPAD), dsts)


# speedup vs baseline: 1.2305x; 1.0482x over previous
import functools

import jax
import jax.numpy as jnp
from jax import lax
from jax.experimental import pallas as pl
from jax.experimental.pallas import tpu as pltpu
from jax.experimental.pallas import tpu_sc as plsc

_NCORE = 2
_NSUB = 16
_K = 80
_HALF = 5120
_NPAD = 10240


def _mm_body(alpha_ref, comp_ref, x_ref, w_ref, upd_ref, o1_ref, o2_ref):
    r = pl.program_id(1)
    x = x_ref[...]

    @pl.when(r < 16)
    def _():
        w = (comp_ref[r, 0] * w_ref[0] + comp_ref[r, 1] * w_ref[1]
             + comp_ref[r, 2] * w_ref[2] + comp_ref[r, 3] * w_ref[3])
        o1_ref[0] = alpha_ref[r, 0] * jnp.dot(
            x, w, preferred_element_type=jnp.float32)

    @pl.when(r == 16)
    def _():
        o2_ref[...] = jnp.dot(
            x, upd_ref[...], preferred_element_type=jnp.float32)


def _build_tables(alphas, comp, vfts, weight, update):
    n, d = vfts.shape
    nrel = comp.shape[0]
    nblk = 10
    b = n // nblk
    return pl.pallas_call(
        _mm_body,
        grid=(nblk, nrel + 1),
        in_specs=[
            pl.BlockSpec((nrel, 1), lambda i, r: (0, 0),
                         memory_space=pltpu.SMEM),
            pl.BlockSpec((nrel, 4), lambda i, r: (0, 0),
                         memory_space=pltpu.SMEM),
            pl.BlockSpec((b, d), lambda i, r: (i, 0)),
            pl.BlockSpec((4, d, d), lambda i, r: (0, 0, 0)),
            pl.BlockSpec((d, d), lambda i, r: (0, 0)),
        ],
        out_specs=[
            pl.BlockSpec((1, b, d), lambda i, r: (jnp.minimum(r, 15), i, 0)),
            pl.BlockSpec((b, d), lambda i, r: (i, 0)),
        ],
        out_shape=[
            jax.ShapeDtypeStruct((nrel, n, d), jnp.float32),
            jax.ShapeDtypeStruct((n, d), jnp.float32),
        ],
        compiler_params=pltpu.CompilerParams(
            dimension_semantics=("arbitrary", "arbitrary")),
    )(alphas, comp, vfts, weight, update)


_SEG = 50


def _make_sc(n, d, e):
    ept = e // _NSUB
    seg = _SEG * _K
    nseg = ept // seg
    rpt = _HALF // _NSUB
    nko = rpt // _K
    mesh = plsc.VectorSubcoreMesh(
        core_axis_name="c", subcore_axis_name="s",
        num_cores=_NCORE, num_subcores=_NSUB)

    @functools.partial(
        pl.kernel,
        out_type=(
            jax.ShapeDtypeStruct((_NCORE, _HALF, d), jnp.float32),
            jax.ShapeDtypeStruct((_NSUB, 1, _NPAD), jnp.float32),
        ),
        mesh=mesh,
        scratch_types=[
            pltpu.VMEM((seg,), jnp.int32),
            pltpu.VMEM((seg,), jnp.int32),
            pltpu.VMEM((seg,), jnp.int32),
            pltpu.VMEM((seg,), jnp.int32),
            pltpu.VMEM((_SEG, _K), jnp.int32),
            pltpu.VMEM((_NPAD,), jnp.float32),
            pltpu.VMEM((2, _K, d), jnp.float32),
            pltpu.VMEM_SHARED((_HALF + 8, d), jnp.float32),
            pltpu.SemaphoreType.DMA,
            pltpu.SemaphoreType.DMA,
        ],
        compiler_params=pltpu.CompilerParams(needs_layout_passes=False),
    )
    def sc_kernel(tbl_hbm, rels_hbm, src_hbm, dst_hbm, zrow_hbm, zcnt_hbm,
                  acc_out, cnt_out,
                  rels_v, src_v, gidx_v, dst1_v, dst2_v, cnt_v, rows_v,
                  acc_sh, sem0, sem1):
        c = lax.axis_index("c")
        s = lax.axis_index("s")
        base_e = s * ept

        pltpu.sync_copy(zcnt_hbm, cnt_v)

        pltpu.sync_copy(zrow_hbm, rows_v.at[1])
        for k in range(nko):
            pltpu.sync_copy(rows_v.at[1],
                            acc_sh.at[pl.ds(s * rpt + k * _K, _K)])

        @pl.when(s == 0)
        def _():
            pltpu.sync_copy(rows_v.at[1, pl.ds(0, 8)],
                            acc_sh.at[pl.ds(_HALF, 8)])

        plsc.subcore_barrier()

        ones16 = jnp.ones((16,), jnp.float32)
        nvec = jnp.full((16,), n, jnp.int32)
        lovec = jnp.zeros((16,), jnp.int32) + c * _HALF
        trash = jnp.full((16,), _HALF, jnp.int32)

        def seg_body(g, carry):
            off = base_e + g * seg
            pltpu.sync_copy(rels_hbm.at[pl.ds(off, seg)], rels_v)
            pltpu.sync_copy(src_hbm.at[pl.ds(off, seg)], src_v)
            pltpu.sync_copy(dst_hbm.at[pl.ds(off, seg)], dst1_v)

            def idx_body(i, cr):
                sl = pl.ds(i * 16, 16)
                gidx_v[sl] = rels_v[sl] * nvec + src_v[sl]
                dv = dst1_v[sl]

                @pl.when(c == 0)
                def _():
                    plsc.addupdate_scatter(cnt_v, [dv], ones16)

                dl = dv - lovec
                oob = (dl < 0) | (dl >= _HALF)
                dst2_v[i // (_K // 16),
                       pl.ds((i % (_K // 16)) * 16, 16)] = jnp.where(
                           oob, trash, dl)
                return cr

            lax.fori_loop(0, seg // 16, idx_body, 0)

            def pair_body(jj, cr):
                j0 = 2 * jj
                j1 = j0 + 1
                g0 = pltpu.async_copy(
                    tbl_hbm.at[gidx_v.at[pl.ds(j0 * _K, _K)]],
                    rows_v.at[0], sem0)
                g1 = pltpu.async_copy(
                    tbl_hbm.at[gidx_v.at[pl.ds(j1 * _K, _K)]],
                    rows_v.at[1], sem1)
                g0.wait()
                pltpu.sync_copy(rows_v.at[0], acc_sh.at[dst2_v.at[j0]],
                                add=True)
                g1.wait()
                pltpu.sync_copy(rows_v.at[1], acc_sh.at[dst2_v.at[j1]],
                                add=True)
                return cr

            lax.fori_loop(0, _SEG // 2, pair_body, 0)
            return carry

        lax.fori_loop(0, nseg, seg_body, 0)

        plsc.subcore_barrier()

        for k in range(nko):
            pltpu.sync_copy(acc_sh.at[pl.ds(s * rpt + k * _K, _K)],
                            rows_v.at[1])
            pltpu.sync_copy(rows_v.at[1],
                            acc_out.at[c, pl.ds(s * rpt + k * _K, _K)])

        @pl.when(c == 0)
        def _():
            pltpu.sync_copy(cnt_v, cnt_out.at[s, 0])

    return sc_kernel


def _combine_body(b, acc_ref, cnt_ref, dst_ref, o_ref):
    i = pl.program_id(0)
    deg = jnp.sum(cnt_ref[:, pl.ds(i * b, b)], axis=0)
    o_ref[...] = jnp.maximum(acc_ref[...] + deg[:, None] * dst_ref[...], 0.0)


def _combine(acc, cnt, dsts):
    n, d = dsts.shape
    b = 512
    nblk = (n + b - 1) // b
    return pl.pallas_call(
        functools.partial(_combine_body, b),
        grid=(nblk,),
        in_specs=[
            pl.BlockSpec((b, d), lambda i: (i, 0)),
            pl.BlockSpec((_NSUB, _NPAD), lambda i: (0, 0)),
            pl.BlockSpec((b, d), lambda i: (i, 0)),
        ],
        out_specs=pl.BlockSpec((b, d), lambda i: (i, 0)),
        out_shape=jax.ShapeDtypeStruct((n, d), jnp.float32),
    )(acc, cnt, dsts)


def kernel(vfts, adjs, rels, embed_rels, embed_rels_target, weight, comp,
           attn_w1, attn_b1, attn_w2, attn_b2, update):
    n, d = vfts.shape
    e = rels.shape[0]
    nrel = comp.shape[0]
    assert e % (_NSUB * _SEG * _K) == 0 and _NCORE * _HALF >= n

    s16 = adjs[0, :nrel]
    t16 = adjs[1, :nrel]
    erps16 = jnp.concatenate(
        [vfts[s16], vfts[t16], embed_rels[:nrel],
         embed_rels_target[:nrel]], axis=1)
    h16 = jax.nn.relu(erps16 @ attn_w1.T + attn_b1)
    alphas = jax.nn.sigmoid(h16 @ attn_w2.T + attn_b2)

    tbl3, dsts = _build_tables(alphas, comp, vfts, weight, update)
    tbl = tbl3.reshape(nrel * n, d)

    zrow = jnp.zeros((_K, d), jnp.float32)
    zcnt = jnp.zeros((_NPAD,), jnp.float32)

    acc, cnt = _make_sc(n, d, e)(tbl, rels, adjs[0], adjs[1], zrow, zcnt)
    return _combine(acc.reshape(_NCORE * _HALF, d),
                    cnt.reshape(_NSUB, _NPAD), dsts)
